# baseline (speedup 1.0000x reference)
import functools

import jax
import jax.numpy as jnp
from jax import lax
from jax.experimental import pallas as pl
from jax.experimental.pallas import tpu as pltpu
from jax.experimental.pallas import tpu_sc as plsc

_S = 256
_H = 128
_NC = 10
_NCORES = 2
_NSUB = 16
_NW = _NCORES * _NSUB


def _reduce_chunk(buf):
    zero = jnp.zeros((16,), jnp.float32)

    def row_body(r, accs):
        al, ar = accs
        for k in range(8):
            al = al + buf[r, pl.ds(k * 16, 16)]
            ar = ar + buf[r, pl.ds(_H + k * 16, 16)]
        return al, ar

    return lax.fori_loop(0, _H, row_body, (zero, zero))


def _sc_body(imgs_per_w, chunk0, x_hbm, m_hbm, cb_hbm, out_hbm,
             buf0, buf1, mv, cbv, outv, sem0, sem1):
    wid = lax.axis_index("s") * _NCORES + lax.axis_index("c")
    chunks_per_w = 2 * imgs_per_w
    base = chunk0 + wid * chunks_per_w
    obase = wid * (imgs_per_w * _NC)

    pltpu.sync_copy(m_hbm, mv)
    pltpu.sync_copy(cb_hbm, cbv)

    pltpu.make_async_copy(x_hbm.at[base], buf0, sem0).start()
    pltpu.make_async_copy(x_hbm.at[base + 1], buf1, sem1).start()

    m0 = mv[0]
    m1 = mv[1]
    m2 = mv[2]
    m3 = mv[3]
    cbvals = cbv[...]
    lanes = lax.iota(jnp.int32, 16)
    omask = lanes < _NC

    def img_body(i, carry):
        c0 = base + 2 * i
        pltpu.make_async_copy(x_hbm.at[c0], buf0, sem0).wait()
        al0, ar0 = _reduce_chunk(buf0)

        @pl.when(i < imgs_per_w - 1)
        def _():
            pltpu.make_async_copy(x_hbm.at[c0 + 2], buf0, sem0).start()

        pltpu.make_async_copy(x_hbm.at[c0 + 1], buf1, sem1).wait()
        al1, ar1 = _reduce_chunk(buf1)

        @pl.when(i < imgs_per_w - 1)
        def _():
            pltpu.make_async_copy(x_hbm.at[c0 + 3], buf1, sem1).start()

        q0 = jnp.sum(al0)
        q3 = jnp.sum(ar0)
        q1 = jnp.sum(al1)
        q2 = jnp.sum(ar1)
        vec = q0 * m0 + q1 * m1 + q2 * m2 + q3 * m3 + cbvals
        plsc.store_scatter(outv, [lanes + i * _NC], vec, mask=omask)
        return 0

    lax.fori_loop(0, imgs_per_w, img_body, 0)
    pltpu.sync_copy(outv, out_hbm.at[pl.ds(obase, imgs_per_w * _NC)])


def _sc_part(x, k_sc, m_pad, cb_pad):
    n = x.shape[0]
    imgs_per_w = k_sc // _NW
    x2 = x.reshape(2 * n, _H, _S)
    mesh = plsc.VectorSubcoreMesh(core_axis_name="c", subcore_axis_name="s")
    out_flat = pl.kernel(
        functools.partial(_sc_body, imgs_per_w, 2 * (n - k_sc)),
        out_type=jax.ShapeDtypeStruct((k_sc * _NC,), jnp.float32),
        mesh=mesh,
        compiler_params=pltpu.CompilerParams(needs_layout_passes=False),
        scratch_types=[
            pltpu.VMEM((_H, _S), jnp.float32),
            pltpu.VMEM((_H, _S), jnp.float32),
            pltpu.VMEM((4, 16), jnp.float32),
            pltpu.VMEM((16,), jnp.float32),
            pltpu.VMEM((imgs_per_w * _NC,), jnp.float32),
            pltpu.SemaphoreType.DMA,
            pltpu.SemaphoreType.DMA,
        ],
    )(x2, m_pad, cb_pad)
    return out_flat.reshape(k_sc, _NC)


def _tc_body(x_ref, m_ref, cb_ref, out_ref):
    xb = x_ref[...]
    tl = jnp.sum(xb[:, :_H, :_H], axis=(1, 2))
    bl = jnp.sum(xb[:, _H:, :_H], axis=(1, 2))
    br = jnp.sum(xb[:, _H:, _H:], axis=(1, 2))
    tr = jnp.sum(xb[:, :_H, _H:], axis=(1, 2))
    m = m_ref[...]
    out_ref[...] = (tl[:, None] * m[0][None, :]
                    + bl[:, None] * m[1][None, :]
                    + br[:, None] * m[2][None, :]
                    + tr[:, None] * m[3][None, :]
                    + cb_ref[...])


_BB = 32


def _tc_part(x, k_sc, m, cb):
    n = x.shape[0]
    return pl.pallas_call(
        _tc_body,
        grid=((n - k_sc) // _BB,),
        in_specs=[
            pl.BlockSpec((_BB, _S, _S), lambda i: (i, 0, 0)),
            pl.BlockSpec((4, _NC), lambda i: (0, 0)),
            pl.BlockSpec((1, _NC), lambda i: (0, 0)),
        ],
        out_specs=pl.BlockSpec((_BB, _NC), lambda i: (i, 0)),
        out_shape=jax.ShapeDtypeStruct((n - k_sc, _NC), jnp.float32),
    )(x, m, cb)


_K_SC = 512


def kernel(x, fgl_v, fgl_g, fgl_b, fc_w, fc_b):
    n = x.shape[0]
    vnorm = jnp.sqrt(jnp.sum(fgl_v ** 2, axis=(1, 2), keepdims=True))
    w = (fgl_g * fgl_v / vnorm).reshape(4, 4)
    fc_w3 = fc_w.reshape(_NC, 4, 4)
    m = jnp.einsum("ij,cij->ic", w, fc_w3)
    m_pad = jnp.zeros((4, 16), jnp.float32).at[:, :_NC].set(m)
    cb = fc_b + jnp.einsum("ij,cij->c", fgl_b, fc_w3)
    cb_pad = jnp.zeros((16,), jnp.float32).at[:_NC].set(cb)

    out_tc = _tc_part(x, _K_SC, m, cb.reshape(1, _NC))
    out_sc = _sc_part(x, _K_SC, m_pad, cb_pad)
    return jnp.concatenate([out_tc, out_sc], axis=0)


# speedup vs baseline: 1.0522x; 1.0522x over previous
import functools

import jax
import jax.numpy as jnp
from jax import lax
from jax.experimental import pallas as pl
from jax.experimental.pallas import tpu as pltpu
from jax.experimental.pallas import tpu_sc as plsc

_S = 256
_H = 128
_NC = 10
_NCORES = 2
_NSUB = 16
_NW = _NCORES * _NSUB


def _reduce_chunk(buf):
    zero = jnp.zeros((16,), jnp.float32)

    def row_body(r, accs):
        al, ar = accs
        for k in range(8):
            al = al + buf[r, pl.ds(k * 16, 16)]
            ar = ar + buf[r, pl.ds(_H + k * 16, 16)]
        return al, ar

    return lax.fori_loop(0, _H, row_body, (zero, zero))


def _sc_body(imgs_per_w, chunk0, x_hbm, m_hbm, cb_hbm, out_hbm,
             buf0, buf1, mv, cbv, outv, sem0, sem1):
    wid = lax.axis_index("s") * _NCORES + lax.axis_index("c")
    chunks_per_w = 2 * imgs_per_w
    base = chunk0 + wid * chunks_per_w
    obase = wid * (imgs_per_w * _NC)

    pltpu.sync_copy(m_hbm, mv)
    pltpu.sync_copy(cb_hbm, cbv)

    pltpu.make_async_copy(x_hbm.at[base], buf0, sem0).start()
    pltpu.make_async_copy(x_hbm.at[base + 1], buf1, sem1).start()

    m0 = mv[0]
    m1 = mv[1]
    m2 = mv[2]
    m3 = mv[3]
    cbvals = cbv[...]
    lanes = lax.iota(jnp.int32, 16)
    omask = lanes < _NC

    def img_body(i, carry):
        c0 = base + 2 * i
        pltpu.make_async_copy(x_hbm.at[c0], buf0, sem0).wait()
        al0, ar0 = _reduce_chunk(buf0)

        @pl.when(i < imgs_per_w - 1)
        def _():
            pltpu.make_async_copy(x_hbm.at[c0 + 2], buf0, sem0).start()

        pltpu.make_async_copy(x_hbm.at[c0 + 1], buf1, sem1).wait()
        al1, ar1 = _reduce_chunk(buf1)

        @pl.when(i < imgs_per_w - 1)
        def _():
            pltpu.make_async_copy(x_hbm.at[c0 + 3], buf1, sem1).start()

        q0 = jnp.sum(al0)
        q3 = jnp.sum(ar0)
        q1 = jnp.sum(al1)
        q2 = jnp.sum(ar1)
        vec = q0 * m0 + q1 * m1 + q2 * m2 + q3 * m3 + cbvals
        plsc.store_scatter(outv, [lanes + i * _NC], vec, mask=omask)
        return 0

    lax.fori_loop(0, imgs_per_w, img_body, 0)
    pltpu.sync_copy(outv, out_hbm.at[pl.ds(obase, imgs_per_w * _NC)])


def _sc_part(x, k_sc, m_pad, cb_pad):
    n = x.shape[0]
    imgs_per_w = k_sc // _NW
    x2 = x.reshape(2 * n, _H, _S)
    mesh = plsc.VectorSubcoreMesh(core_axis_name="c", subcore_axis_name="s")
    out_flat = pl.kernel(
        functools.partial(_sc_body, imgs_per_w, 2 * (n - k_sc)),
        out_type=jax.ShapeDtypeStruct((k_sc * _NC,), jnp.float32),
        mesh=mesh,
        compiler_params=pltpu.CompilerParams(needs_layout_passes=False),
        scratch_types=[
            pltpu.VMEM((_H, _S), jnp.float32),
            pltpu.VMEM((_H, _S), jnp.float32),
            pltpu.VMEM((4, 16), jnp.float32),
            pltpu.VMEM((16,), jnp.float32),
            pltpu.VMEM((imgs_per_w * _NC,), jnp.float32),
            pltpu.SemaphoreType.DMA,
            pltpu.SemaphoreType.DMA,
        ],
    )(x2, m_pad, cb_pad)
    return out_flat.reshape(k_sc, _NC)


def _tc_body(x_ref, m_ref, cb_ref, out_ref):
    xb = x_ref[...]
    tl = jnp.sum(xb[:, :_H, :_H], axis=(1, 2))
    bl = jnp.sum(xb[:, _H:, :_H], axis=(1, 2))
    br = jnp.sum(xb[:, _H:, _H:], axis=(1, 2))
    tr = jnp.sum(xb[:, :_H, _H:], axis=(1, 2))
    m = m_ref[...]
    out_ref[...] = (tl[:, None] * m[0][None, :]
                    + bl[:, None] * m[1][None, :]
                    + br[:, None] * m[2][None, :]
                    + tr[:, None] * m[3][None, :]
                    + cb_ref[...])


_BB = 32


def _tc_part(x, k_sc, m, cb):
    n = x.shape[0]
    return pl.pallas_call(
        _tc_body,
        grid=((n - k_sc) // _BB,),
        in_specs=[
            pl.BlockSpec((_BB, _S, _S), lambda i: (i, 0, 0)),
            pl.BlockSpec((4, _NC), lambda i: (0, 0)),
            pl.BlockSpec((1, _NC), lambda i: (0, 0)),
        ],
        out_specs=pl.BlockSpec((_BB, _NC), lambda i: (i, 0)),
        out_shape=jax.ShapeDtypeStruct((n - k_sc, _NC), jnp.float32),
    )(x, m, cb)


_K_SC = 128


def kernel(x, fgl_v, fgl_g, fgl_b, fc_w, fc_b):
    n = x.shape[0]
    vnorm = jnp.sqrt(jnp.sum(fgl_v ** 2, axis=(1, 2), keepdims=True))
    w = (fgl_g * fgl_v / vnorm).reshape(4, 4)
    fc_w3 = fc_w.reshape(_NC, 4, 4)
    m = jnp.einsum("ij,cij->ic", w, fc_w3)
    m_pad = jnp.zeros((4, 16), jnp.float32).at[:, :_NC].set(m)
    cb = fc_b + jnp.einsum("ij,cij->c", fgl_b, fc_w3)
    cb_pad = jnp.zeros((16,), jnp.float32).at[:_NC].set(cb)

    out_tc = _tc_part(x, _K_SC, m, cb.reshape(1, _NC))
    out_sc = _sc_part(x, _K_SC, m_pad, cb_pad)
    return jnp.concatenate([out_tc, out_sc], axis=0)
